# baseline (speedup 1.0000x reference)
import functools

import jax
import jax.numpy as jnp
from jax import lax
from jax.experimental import pallas as pl
from jax.experimental.pallas import tpu as pltpu
from jax.experimental.pallas import tpu_sc as plsc

N_NODES = 10000
N_PAD = 10240
N_EDGES = 320000
NUM_CORES = 2
NUM_SUBCORES = 16
NW = NUM_CORES * NUM_SUBCORES
E_STRIDE = 9984
E_CHUNK = 10496
COL_W = N_PAD // NUM_SUBCORES
LEAKY_ALPHA = 0.1

def _nbr_body(edges_hbm, curr_hbm, out_hbm, ev_v, curr_v, ind_v, acc_v, tmp_v, shared_v):
    wid = lax.axis_index("s") * NUM_CORES + lax.axis_index("c")
    base = pl.multiple_of(wid * E_STRIDE, 128)
    pltpu.sync_copy(edges_hbm.at[:, pl.ds(base, E_CHUNK)], ev_v)
    pltpu.sync_copy(curr_hbm, curr_v)

    @plsc.parallel_loop(0, N_PAD, step=16, unroll=8)
    def _zero(i):
        ind_v[pl.ds(i, 16)] = jnp.zeros((16,), jnp.float32)

    curr16 = curr_v[...]
    ones16 = jnp.ones((16,), jnp.float32)

    @plsc.parallel_loop(0, E_CHUNK, step=16, unroll=8)
    def _edges(i):
        s16 = ev_v[0, pl.ds(i, 16)]
        d16 = ev_v[1, pl.ds(i, 16)]
        plsc.store_scatter(ind_v, [d16], ones16, mask=s16 == curr16)

    sid = lax.axis_index("s")
    cid = lax.axis_index("c")
    pltpu.sync_copy(ind_v, shared_v.at[sid])
    plsc.subcore_barrier()
    col = pl.multiple_of(sid * COL_W, 128)

    @plsc.parallel_loop(0, COL_W, step=16, unroll=8)
    def _zacc(i):
        acc_v[pl.ds(i, 16)] = jnp.zeros((16,), jnp.float32)

    def _row(r, carry):
        pltpu.sync_copy(shared_v.at[r, pl.ds(col, COL_W)], tmp_v)

        @plsc.parallel_loop(0, COL_W, step=16, unroll=8)
        def _acc(i):
            acc_v[pl.ds(i, 16)] = acc_v[pl.ds(i, 16)] + tmp_v[pl.ds(i, 16)]

        return carry

    lax.fori_loop(0, NUM_SUBCORES, _row, 0)
    pltpu.sync_copy(acc_v, out_hbm.at[cid, pl.ds(col, COL_W)])


@functools.lru_cache(maxsize=1)
def _nbr_counts_kernel():
    return pl.kernel(
        _nbr_body,
        mesh=plsc.VectorSubcoreMesh(core_axis_name="c", subcore_axis_name="s"),
        compiler_params=pltpu.CompilerParams(needs_layout_passes=False),
        out_type=jax.ShapeDtypeStruct((NUM_CORES, N_PAD), jnp.float32),
        scratch_types=[
            pltpu.VMEM((2, E_CHUNK), jnp.int32),
            pltpu.VMEM((16,), jnp.int32),
            pltpu.VMEM((N_PAD,), jnp.float32),
            pltpu.VMEM((COL_W,), jnp.float32),
            pltpu.VMEM((COL_W,), jnp.float32),
            pltpu.VMEM_SHARED((NUM_SUBCORES, N_PAD), jnp.float32),
        ],
    )


def _mlp_scores(x_ref, w1_ref, b1_ref, w2_ref, b2_ref, out_ref):
    h = jnp.tanh(x_ref[...])
    hid_t = lax.dot_general(w1_ref[...], h, (((0,), (1,)), ((), ())),
                            preferred_element_type=jnp.float32)
    hid_t = hid_t + b1_ref[...]
    hid_t = jnp.where(hid_t > 0, hid_t, LEAKY_ALPHA * hid_t)
    out_ref[...] = jnp.sum(hid_t * w2_ref[...], axis=0, keepdims=True) + b2_ref[...]


def _masked_softmax(s_ref, counts_ref, out_ref):
    deg = jnp.sum(counts_ref[...], axis=0, keepdims=True)[:, :N_NODES]
    masked = jnp.where(deg > 0, s_ref[...], -1e9)
    m = jnp.max(masked)
    e = jnp.exp(masked - m)
    out_ref[...] = (e / jnp.sum(e))[0]


def kernel(x, edge_index, current_vertex_idx, Wc1, bc1, Wc2, bc2, Wc3, bc3,
           W1, b1, W2, b2):
    curr = jnp.full((16,), current_vertex_idx, jnp.int32)
    counts = _nbr_counts_kernel()(edge_index.astype(jnp.int32), curr)
    scores = pl.pallas_call(
        _mlp_scores,
        out_shape=jax.ShapeDtypeStruct((1, N_NODES), jnp.float32),
    )(x, W1, b1.reshape(-1, 1), W2, b2.reshape(1, 1))
    return pl.pallas_call(
        _masked_softmax,
        out_shape=jax.ShapeDtypeStruct((N_NODES,), jnp.float32),
    )(scores, counts)


# speedup vs baseline: 1.0003x; 1.0003x over previous
import functools

import jax
import jax.numpy as jnp
from jax import lax
from jax.experimental import pallas as pl
from jax.experimental.pallas import tpu as pltpu
from jax.experimental.pallas import tpu_sc as plsc

N_NODES = 10000
N_PAD = 10240
N_EDGES = 320000
NUM_CORES = 2
NUM_SUBCORES = 16
NW = NUM_CORES * NUM_SUBCORES
E_STRIDE = 9984
E_CHUNK = 10496
COL_W = N_PAD // NUM_SUBCORES
LEAKY_ALPHA = 0.1

def _nbr_body(edges_hbm, curr_hbm, out_hbm, ev_v, curr_v, ind_v):
    wid = lax.axis_index("s") * NUM_CORES + lax.axis_index("c")
    base = pl.multiple_of(wid * E_STRIDE, 128)
    pltpu.sync_copy(edges_hbm.at[:, pl.ds(base, E_CHUNK)], ev_v)
    pltpu.sync_copy(curr_hbm, curr_v)

    @plsc.parallel_loop(0, N_PAD, step=16, unroll=8)
    def _zero(i):
        ind_v[pl.ds(i, 16)] = jnp.zeros((16,), jnp.float32)

    curr16 = curr_v[...]
    ones16 = jnp.ones((16,), jnp.float32)

    @plsc.parallel_loop(0, E_CHUNK, step=16, unroll=16)
    def _edges(i):
        s16 = ev_v[0, pl.ds(i, 16)]
        d16 = ev_v[1, pl.ds(i, 16)]
        plsc.store_scatter(ind_v, [d16], ones16, mask=s16 == curr16)

    pltpu.sync_copy(ind_v, out_hbm.at[wid])


@functools.lru_cache(maxsize=1)
def _nbr_counts_kernel():
    return pl.kernel(
        _nbr_body,
        mesh=plsc.VectorSubcoreMesh(core_axis_name="c", subcore_axis_name="s"),
        compiler_params=pltpu.CompilerParams(needs_layout_passes=False),
        out_type=jax.ShapeDtypeStruct((NW, N_PAD), jnp.float32),
        scratch_types=[
            pltpu.VMEM((2, E_CHUNK), jnp.int32),
            pltpu.VMEM((16,), jnp.int32),
            pltpu.VMEM((N_PAD,), jnp.float32),
        ],
    )


def _mlp_scores(x_ref, w1_ref, b1_ref, w2_ref, b2_ref, out_ref):
    blk = x_ref.shape[0]
    h = jnp.tanh(x_ref[...])
    hid_t = lax.dot_general(w1_ref[...], h, (((0,), (1,)), ((), ())),
                            preferred_element_type=jnp.float32)
    hid_t = hid_t + b1_ref[...]
    hid_t = jnp.where(hid_t > 0, hid_t, LEAKY_ALPHA * hid_t)
    s = jnp.sum(hid_t * w2_ref[...], axis=0, keepdims=True) + b2_ref[...]
    i = pl.program_id(0)
    out_ref[:, pl.ds(i * blk, blk)] = s


def _masked_softmax(s_ref, counts_ref, out_ref):
    deg = jnp.sum(counts_ref[...], axis=0, keepdims=True)[:, :N_NODES]
    masked = jnp.where(deg > 0, s_ref[...][:, :N_NODES], -1e9)
    m = jnp.max(masked)
    e = jnp.exp(masked - m)
    out_ref[...] = (e / jnp.sum(e))[0]


def kernel(x, edge_index, current_vertex_idx, Wc1, bc1, Wc2, bc2, Wc3, bc3,
           W1, b1, W2, b2):
    curr = jnp.full((16,), current_vertex_idx, jnp.int32)
    counts = _nbr_counts_kernel()(edge_index.astype(jnp.int32), curr)
    blk = 1280
    scores = pl.pallas_call(
        _mlp_scores,
        grid=(N_PAD // blk,),
        in_specs=[
            pl.BlockSpec((blk, 128), lambda i: (i, 0)),
            pl.BlockSpec((128, 256), lambda i: (0, 0)),
            pl.BlockSpec((256, 1), lambda i: (0, 0)),
            pl.BlockSpec((256, 1), lambda i: (0, 0)),
            pl.BlockSpec((1, 1), lambda i: (0, 0)),
        ],
        out_specs=pl.BlockSpec((1, N_PAD), lambda i: (0, 0)),
        out_shape=jax.ShapeDtypeStruct((1, N_PAD), jnp.float32),
    )(x, W1, b1.reshape(-1, 1), W2, b2.reshape(1, 1))
    return pl.pallas_call(
        _masked_softmax,
        out_shape=jax.ShapeDtypeStruct((N_NODES,), jnp.float32),
    )(scores, counts)


# speedup vs baseline: 1.0462x; 1.0458x over previous
import functools

import jax
import jax.numpy as jnp
from jax import lax
from jax.experimental import pallas as pl
from jax.experimental.pallas import tpu as pltpu
from jax.experimental.pallas import tpu_sc as plsc

N_NODES = 10000
N_PAD = 10240
N_EDGES = 320000
NUM_CORES = 2
NUM_SUBCORES = 16
NW = NUM_CORES * NUM_SUBCORES
E_STRIDE = 9984
E_CHUNK = 10496
COL_W = N_PAD // NUM_SUBCORES
LEAKY_ALPHA = 0.1

def _nbr_body(edges_hbm, curr_hbm, out_hbm, ev_v, curr_v, ind_v):
    wid = lax.axis_index("s") * NUM_CORES + lax.axis_index("c")
    base = pl.multiple_of(wid * E_STRIDE, 128)
    pltpu.sync_copy(edges_hbm.at[:, pl.ds(base, E_CHUNK)], ev_v)
    pltpu.sync_copy(curr_hbm, curr_v)

    @plsc.parallel_loop(0, N_PAD, step=16, unroll=4)
    def _zero(i):
        ind_v[pl.ds(i, 16)] = jnp.zeros((16,), jnp.float32)

    curr16 = curr_v[...]
    ones16 = jnp.ones((16,), jnp.float32)

    @plsc.parallel_loop(0, E_CHUNK, step=16, unroll=4)
    def _edges(i):
        s16 = ev_v[0, pl.ds(i, 16)]
        d16 = ev_v[1, pl.ds(i, 16)]
        plsc.store_scatter(ind_v, [d16], ones16, mask=s16 == curr16)

    pltpu.sync_copy(ind_v, out_hbm.at[wid])


@functools.lru_cache(maxsize=1)
def _nbr_counts_kernel():
    return pl.kernel(
        _nbr_body,
        mesh=plsc.VectorSubcoreMesh(core_axis_name="c", subcore_axis_name="s"),
        compiler_params=pltpu.CompilerParams(needs_layout_passes=False),
        out_type=jax.ShapeDtypeStruct((NW, N_PAD), jnp.float32),
        scratch_types=[
            pltpu.VMEM((2, E_CHUNK), jnp.int32),
            pltpu.VMEM((16,), jnp.int32),
            pltpu.VMEM((N_PAD,), jnp.float32),
        ],
    )


def _mlp_scores(x_ref, w1_ref, b1_ref, w2_ref, b2_ref, out_ref):
    h = jnp.tanh(x_ref[...])
    hid_t = lax.dot_general(w1_ref[...], h, (((0,), (1,)), ((), ())),
                            preferred_element_type=jnp.float32)
    hid_t = hid_t + b1_ref[...]
    hid_t = jnp.where(hid_t > 0, hid_t, LEAKY_ALPHA * hid_t)
    out_ref[...] = jnp.sum(hid_t * w2_ref[...], axis=0, keepdims=True) + b2_ref[...]


def _masked_softmax(s_ref, counts_ref, out_ref):
    deg = jnp.sum(counts_ref[...], axis=0, keepdims=True)[:, :N_NODES]
    masked = jnp.where(deg > 0, s_ref[...], -1e9)
    m = jnp.max(masked)
    e = jnp.exp(masked - m)
    out_ref[...] = (e / jnp.sum(e))[0]


def kernel(x, edge_index, current_vertex_idx, Wc1, bc1, Wc2, bc2, Wc3, bc3,
           W1, b1, W2, b2):
    curr = jnp.full((16,), current_vertex_idx, jnp.int32)
    counts = _nbr_counts_kernel()(edge_index.astype(jnp.int32), curr)
    scores = pl.pallas_call(
        _mlp_scores,
        out_shape=jax.ShapeDtypeStruct((1, N_NODES), jnp.float32),
    )(x, W1, b1.reshape(-1, 1), W2, b2.reshape(1, 1))
    return pl.pallas_call(
        _masked_softmax,
        out_shape=jax.ShapeDtypeStruct((N_NODES,), jnp.float32),
    )(scores, counts)
